# baseline (speedup 1.0000x reference)
import functools

import jax
import jax.numpy as jnp
from jax import lax
from jax.experimental import pallas as pl
from jax.experimental.pallas import tpu as pltpu
from jax.experimental.pallas import tpu_sc as plsc

_NC = 2
_NS = 16
_NW = _NC * _NS
_L = 16


def _round_up(v, m):
    return (v + m - 1) // m * m


def _make_proj_body(npad):
    def _proj_body(x_ref, we_ref, wg_ref, be_ref, a_ref, b_ref, h_ref):
        x = x_ref[...]
        n, d = x.shape
        pad = ((0, npad - n), (0, 0))
        a = jnp.dot(x, we_ref[0:d, :], preferred_element_type=jnp.float32)
        b = jnp.dot(x, we_ref[d:2 * d, :], preferred_element_type=jnp.float32)
        h = jnp.dot(x, wg_ref[...], preferred_element_type=jnp.float32)
        a_ref[...] = jnp.pad(a + be_ref[...], pad)
        b_ref[...] = jnp.pad(b, pad)
        h_ref[...] = jnp.pad(h, pad)

    return _proj_body


def _make_soft_body(n, e):
    def _soft_body(dis_ref, g_ref, sp_ref, split_ref, bg_ref,
                   eam_ref, eas_ref, nm_ref, ns_ref, eamo_ref, easo_ref):
        eamo_ref[...] = eam_ref[0:e]
        easo_ref[...] = eas_ref[0:e]
        npad = dis_ref.shape[0]
        nb = split_ref.shape[0]
        dis = dis_ref[...]
        g = g_ref[...]
        s = sp_ref[0, :] + sp_ref[1, :]
        natt = dis * (s + g) + bg_ref[...]

        spl = split_ref[...]
        ib = lax.broadcasted_iota(jnp.int32, (nb, nb), 0)
        jb = lax.broadcasted_iota(jnp.int32, (nb, nb), 1)
        oincl = jnp.sum(jnp.where(jb <= ib, spl[None, :], 0), axis=1)
        oexcl = oincl - spl
        ii = lax.broadcasted_iota(jnp.int32, (nb, npad), 1)
        mask = (ii >= oexcl[:, None]) & (ii < oincl[:, None])

        def segsoft(v):
            m = jnp.max(jnp.where(mask, v[None, :], -jnp.inf), axis=1)
            mn = jnp.sum(jnp.where(mask, m[:, None], 0.0), axis=0)
            e = jnp.exp(v - mn)
            sb = jnp.sum(jnp.where(mask, e[None, :], 0.0), axis=1)
            sn = jnp.sum(jnp.where(mask, sb[:, None], 0.0), axis=0)
            return e / jnp.maximum(sn, 1e-16)

        nm = segsoft(natt)
        nm_ref[...] = nm[:n]
        ns_ref[...] = segsoft(1.0 - nm)[:n]

    return _soft_body


def _make_mega_kernel(ep, ew, npad):
    vecs = ew // _L
    nvec = npad // _L
    nsl = npad // _NS
    nslv = nsl // _L
    mesh = plsc.VectorSubcoreMesh(core_axis_name="c", subcore_axis_name="s")

    @functools.partial(
        pl.kernel,
        out_type=[
            jax.ShapeDtypeStruct((ep,), jnp.float32),
            jax.ShapeDtypeStruct((ep,), jnp.float32),
            jax.ShapeDtypeStruct((_NC, npad), jnp.float32),
            jax.ShapeDtypeStruct((npad,), jnp.float32),
            jax.ShapeDtypeStruct((npad,), jnp.float32),
        ],
        mesh=mesh,
        scratch_types=[
            pltpu.VMEM((ew,), jnp.int32),
            pltpu.VMEM((ew,), jnp.int32),
            pltpu.VMEM((ew,), jnp.int32),
            pltpu.VMEM((npad,), jnp.float32),
            pltpu.VMEM((npad,), jnp.float32),
            pltpu.VMEM((npad,), jnp.float32),
            pltpu.VMEM((ew,), jnp.float32),
            pltpu.VMEM((ew,), jnp.float32),
            pltpu.VMEM((ew,), jnp.float32),
            pltpu.VMEM((ew,), jnp.float32),
            pltpu.VMEM((npad,), jnp.float32),
            pltpu.VMEM((nsl,), jnp.float32),
            pltpu.VMEM((nsl,), jnp.float32),
            pltpu.VMEM((nsl,), jnp.float32),
            pltpu.VMEM_SHARED((npad,), jnp.float32),
            pltpu.VMEM_SHARED((npad,), jnp.float32),
            pltpu.VMEM_SHARED((npad,), jnp.float32),
            pltpu.SemaphoreType.DMA,
            pltpu.SemaphoreType.DMA,
            pltpu.SemaphoreType.DMA,
            pltpu.SemaphoreType.DMA,
            pltpu.SemaphoreType.DMA,
            pltpu.SemaphoreType.DMA,
            pltpu.SemaphoreType.DMA,
            pltpu.SemaphoreType.DMA,
            pltpu.SemaphoreType.DMA,
            pltpu.SemaphoreType.DMA,
            pltpu.SemaphoreType.DMA,
        ],
        compiler_params=pltpu.CompilerParams(needs_layout_passes=False),
    )
    def mk(row_h, col_h, a_h, b_h, h_h, eam_h, eas_h, sp_h, dis_h, g_h,
           row1_v, row2_v, col_v, a_v, b_v, g_v, m_v, s_v, val_v,
           one_v, zero_v, d_sl, h_sl, g_sl, acc_deg, acc_s, g_sh,
           sem_r1, sem_r2, sem_c, sem_a, sem_b, sem_h,
           sem_d1, sem_d2, sem_sc, sem_m, sem_s):
        c = lax.axis_index("c")
        s = lax.axis_index("s")
        w_own = 2 * s + c
        w_oth = 2 * s + 1 - c
        base_own = w_own * ew
        base_oth = w_oth * ew
        nbase = s * nsl
        dr1 = pltpu.async_copy(row_h.at[pl.ds(base_own, ew)], row1_v, sem_r1)
        dr2 = pltpu.async_copy(row_h.at[pl.ds(base_oth, ew)], row2_v, sem_r2)
        dc = pltpu.async_copy(col_h.at[pl.ds(base_own, ew)], col_v, sem_c)
        da = pltpu.async_copy(a_h, a_v, sem_a)
        db = pltpu.async_copy(b_h, b_v, sem_b)
        dh = pltpu.async_copy(h_h.at[pl.ds(nbase, nsl)], h_sl, sem_h)
        ones = jnp.full((_L,), 1.0, jnp.float32)
        zeros = jnp.zeros((_L,), jnp.float32)

        @plsc.parallel_loop(0, vecs, unroll=8)
        def _(i):
            one_v[pl.ds(i * _L, _L)] = ones

        @pl.when(s == 0)
        def _():
            @plsc.parallel_loop(0, nvec, unroll=8)
            def _(i):
                zero_v[pl.ds(i * _L, _L)] = zeros

            pltpu.sync_copy(zero_v, acc_deg)
            pltpu.sync_copy(zero_v, acc_s)

        plsc.subcore_barrier()
        dr1.wait()
        dsc1 = pltpu.async_copy(one_v, acc_deg.at[row1_v], sem_d1, add=True)
        dr2.wait()
        dsc2 = pltpu.async_copy(one_v, acc_deg.at[row2_v], sem_d2, add=True)
        da.wait()
        db.wait()
        dc.wait()

        @plsc.parallel_loop(0, vecs, unroll=4)
        def _(i):
            sl = pl.ds(i * _L, _L)
            av = plsc.load_gather(a_v, [row1_v[sl]])
            bv = plsc.load_gather(b_v, [col_v[sl]])
            m = 1.0 / (1.0 + jnp.exp(-(av + bv)))
            m_v[sl] = m
            s_v[sl] = 1.0 - m

        dm = pltpu.async_copy(m_v, eam_h.at[pl.ds(base_own, ew)], sem_m)
        ds2 = pltpu.async_copy(s_v, eas_h.at[pl.ds(base_own, ew)], sem_s)
        dsc1.wait()
        dsc2.wait()
        plsc.subcore_barrier()
        pltpu.sync_copy(acc_deg.at[pl.ds(nbase, nsl)], d_sl)
        dh.wait()
        magic = jnp.full((_L,), 0x5F3759DF, jnp.int32)

        @plsc.parallel_loop(0, nslv, unroll=4)
        def _(i):
            sl = pl.ds(i * _L, _L)
            deg = d_sl[sl] + 1.0
            y = plsc.bitcast(
                magic - lax.shift_right_logical(plsc.bitcast(deg, jnp.int32), 1),
                jnp.float32,
            )
            hd = -0.5 * deg
            y = y * (1.5 + hd * y * y)
            y = y * (1.5 + hd * y * y)
            y = y * (1.5 + hd * y * y)
            d_sl[sl] = y
            g_sl[sl] = y * h_sl[sl]

        pltpu.sync_copy(g_sl, g_sh.at[pl.ds(nbase, nsl)])

        @pl.when(c == 0)
        def _():
            pltpu.sync_copy(d_sl, dis_h.at[pl.ds(nbase, nsl)])
            pltpu.sync_copy(g_sl, g_h.at[pl.ds(nbase, nsl)])

        plsc.subcore_barrier()
        pltpu.sync_copy(g_sh, g_v)

        @plsc.parallel_loop(0, vecs, unroll=4)
        def _(i):
            sl = pl.ds(i * _L, _L)
            val_v[sl] = plsc.load_gather(g_v, [row1_v[sl]])

        dscs = pltpu.async_copy(val_v, acc_s.at[col_v], sem_sc, add=True)
        dscs.wait()
        dm.wait()
        ds2.wait()
        plsc.subcore_barrier()

        @pl.when(s == 0)
        def _():
            pltpu.sync_copy(acc_s, sp_h.at[c])

    return mk


def kernel(x, edge_index, split_n, We, be, Wg, bg):
    n, d = x.shape
    e = edge_index.shape[1]
    npad = _round_up(n, _NS * 128)
    ew = _round_up(e, _NW * _L) // _NW
    ep = ew * _NW

    a2, b2, h2 = pl.pallas_call(
        _make_proj_body(npad),
        out_shape=[
            jax.ShapeDtypeStruct((npad, 1), jnp.float32),
            jax.ShapeDtypeStruct((npad, 1), jnp.float32),
            jax.ShapeDtypeStruct((npad, 1), jnp.float32),
        ],
    )(x, We, Wg, be)
    a = a2.reshape(npad)
    b = b2.reshape(npad)
    h = h2.reshape(npad)

    pad_e = jnp.full((ep - e,), n, dtype=jnp.int32)
    rowp = jnp.concatenate([edge_index[0], pad_e])
    colp = jnp.concatenate([edge_index[1], pad_e])

    eam, eas, sp, dis, g = _make_mega_kernel(ep, ew, npad)(rowp, colp, a, b, h)

    nm, ns, eamo, easo = pl.pallas_call(
        _make_soft_body(n, e),
        out_shape=[
            jax.ShapeDtypeStruct((n,), jnp.float32),
            jax.ShapeDtypeStruct((n,), jnp.float32),
            jax.ShapeDtypeStruct((e,), jnp.float32),
            jax.ShapeDtypeStruct((e,), jnp.float32),
        ],
    )(dis, g, sp, split_n, bg, eam, eas)

    return (
        eamo.reshape(e, 1),
        easo.reshape(e, 1),
        nm.reshape(n, 1),
        ns.reshape(n, 1),
    )


# speedup vs baseline: 1.3112x; 1.3112x over previous
import functools

import jax
import jax.numpy as jnp
from jax import lax
from jax.experimental import pallas as pl
from jax.experimental.pallas import tpu as pltpu
from jax.experimental.pallas import tpu_sc as plsc

_NC = 2
_NS = 16
_NW = _NC * _NS
_L = 16


def _round_up(v, m):
    return (v + m - 1) // m * m


def _make_proj_body(npad):
    def _proj_body(x_ref, we_ref, wg_ref, be_ref, p_ref):
        x = x_ref[...]
        n, d = x.shape
        w3 = jnp.concatenate(
            [we_ref[0:d, :], we_ref[d:2 * d, :], wg_ref[...]], axis=1
        )
        p = lax.dot_general(
            w3, x, (((0,), (1,)), ((), ())),
            preferred_element_type=jnp.float32,
        )
        bias = jnp.pad(be_ref[...][None, :], ((0, 2), (0, 0)))
        p_ref[...] = jnp.pad(p + bias, ((0, 0), (0, npad - n)))

    return _proj_body


def _make_soft_body(n, e):
    def _soft_body(dis_ref, g_ref, sp_ref, split_ref, bg_ref,
                   eam_ref, eas_ref, nm_ref, ns_ref, eamo_ref, easo_ref):
        eamo_ref[...] = eam_ref[0:e]
        easo_ref[...] = eas_ref[0:e]
        npad = dis_ref.shape[0]
        nb = split_ref.shape[0]
        dis = dis_ref[...]
        g = g_ref[...]
        s = sp_ref[0, :] + sp_ref[1, :]
        natt = dis * (s + g) + bg_ref[...]

        spl = split_ref[...]
        ib = lax.broadcasted_iota(jnp.int32, (nb, nb), 0)
        jb = lax.broadcasted_iota(jnp.int32, (nb, nb), 1)
        oincl = jnp.sum(jnp.where(jb <= ib, spl[None, :], 0), axis=1)
        oexcl = oincl - spl
        ii = lax.broadcasted_iota(jnp.int32, (nb, npad), 1)
        mask = (ii >= oexcl[:, None]) & (ii < oincl[:, None])

        def segsoft(v):
            m = jnp.max(jnp.where(mask, v[None, :], -jnp.inf), axis=1)
            mn = jnp.sum(jnp.where(mask, m[:, None], 0.0), axis=0)
            e = jnp.exp(v - mn)
            sb = jnp.sum(jnp.where(mask, e[None, :], 0.0), axis=1)
            sn = jnp.sum(jnp.where(mask, sb[:, None], 0.0), axis=0)
            return e / jnp.maximum(sn, 1e-16)

        nm = segsoft(natt)
        nm_ref[...] = nm[:n]
        ns_ref[...] = segsoft(1.0 - nm)[:n]

    return _soft_body


def _make_mega_kernel(ep, ew, npad):
    vecs = ew // _L
    nvec = npad // _L
    nsl = npad // _NS
    nslv = nsl // _L
    mesh = plsc.VectorSubcoreMesh(core_axis_name="c", subcore_axis_name="s")

    @functools.partial(
        pl.kernel,
        out_type=[
            jax.ShapeDtypeStruct((ep,), jnp.float32),
            jax.ShapeDtypeStruct((ep,), jnp.float32),
            jax.ShapeDtypeStruct((_NC, npad), jnp.float32),
            jax.ShapeDtypeStruct((npad,), jnp.float32),
            jax.ShapeDtypeStruct((npad,), jnp.float32),
        ],
        mesh=mesh,
        scratch_types=[
            pltpu.VMEM((ew,), jnp.int32),
            pltpu.VMEM((ew,), jnp.int32),
            pltpu.VMEM((ew,), jnp.int32),
            pltpu.VMEM((npad,), jnp.float32),
            pltpu.VMEM((npad,), jnp.float32),
            pltpu.VMEM((npad,), jnp.float32),
            pltpu.VMEM((ew,), jnp.float32),
            pltpu.VMEM((ew,), jnp.float32),
            pltpu.VMEM((ew,), jnp.float32),
            pltpu.VMEM((ew,), jnp.float32),
            pltpu.VMEM((npad,), jnp.float32),
            pltpu.VMEM((nsl,), jnp.float32),
            pltpu.VMEM((nsl,), jnp.float32),
            pltpu.VMEM((nsl,), jnp.float32),
            pltpu.VMEM_SHARED((npad,), jnp.float32),
            pltpu.VMEM_SHARED((npad,), jnp.float32),
            pltpu.VMEM_SHARED((npad,), jnp.float32),
            pltpu.SemaphoreType.DMA,
            pltpu.SemaphoreType.DMA,
            pltpu.SemaphoreType.DMA,
            pltpu.SemaphoreType.DMA,
            pltpu.SemaphoreType.DMA,
            pltpu.SemaphoreType.DMA,
            pltpu.SemaphoreType.DMA,
            pltpu.SemaphoreType.DMA,
            pltpu.SemaphoreType.DMA,
            pltpu.SemaphoreType.DMA,
            pltpu.SemaphoreType.DMA,
        ],
        compiler_params=pltpu.CompilerParams(needs_layout_passes=False),
    )
    def mk(ei_h, p3_h, eam_h, eas_h, sp_h, dis_h, g_h,
           row1_v, row2_v, col_v, a_v, b_v, g_v, m_v, s_v, val_v,
           one_v, zero_v, d_sl, h_sl, g_sl, acc_deg, acc_s, g_sh,
           sem_r1, sem_r2, sem_c, sem_a, sem_b, sem_h,
           sem_d1, sem_d2, sem_sc, sem_m, sem_s):
        c = lax.axis_index("c")
        s = lax.axis_index("s")
        w_own = 2 * s + c
        w_oth = 2 * s + 1 - c
        base_own = w_own * ew
        base_oth = w_oth * ew
        nbase = s * nsl
        dr1 = pltpu.async_copy(ei_h.at[0].at[pl.ds(base_own, ew)], row1_v, sem_r1)
        dr2 = pltpu.async_copy(ei_h.at[0].at[pl.ds(base_oth, ew)], row2_v, sem_r2)
        dc = pltpu.async_copy(ei_h.at[1].at[pl.ds(base_own, ew)], col_v, sem_c)
        da = pltpu.async_copy(p3_h.at[pl.ds(0, npad)], a_v, sem_a)
        db = pltpu.async_copy(p3_h.at[pl.ds(npad, npad)], b_v, sem_b)
        dh = pltpu.async_copy(p3_h.at[pl.ds(2 * npad + nbase, nsl)], h_sl, sem_h)
        ones = jnp.full((_L,), 1.0, jnp.float32)
        zeros = jnp.zeros((_L,), jnp.float32)

        @plsc.parallel_loop(0, vecs, unroll=8)
        def _(i):
            one_v[pl.ds(i * _L, _L)] = ones

        @pl.when(s == 0)
        def _():
            @plsc.parallel_loop(0, nvec, unroll=8)
            def _(i):
                zero_v[pl.ds(i * _L, _L)] = zeros

            pltpu.sync_copy(zero_v, acc_deg)
            pltpu.sync_copy(zero_v, acc_s)

        plsc.subcore_barrier()
        dr1.wait()
        dsc1 = pltpu.async_copy(one_v, acc_deg.at[row1_v], sem_d1, add=True)
        dr2.wait()
        dsc2 = pltpu.async_copy(one_v, acc_deg.at[row2_v], sem_d2, add=True)
        da.wait()
        db.wait()
        dc.wait()

        @plsc.parallel_loop(0, vecs, unroll=4)
        def _(i):
            sl = pl.ds(i * _L, _L)
            av = plsc.load_gather(a_v, [row1_v[sl]])
            bv = plsc.load_gather(b_v, [col_v[sl]])
            m = 1.0 / (1.0 + jnp.exp(-(av + bv)))
            m_v[sl] = m
            s_v[sl] = 1.0 - m

        dm = pltpu.async_copy(m_v, eam_h.at[pl.ds(base_own, ew)], sem_m)
        ds2 = pltpu.async_copy(s_v, eas_h.at[pl.ds(base_own, ew)], sem_s)
        dsc1.wait()
        dsc2.wait()
        plsc.subcore_barrier()
        pltpu.sync_copy(acc_deg.at[pl.ds(nbase, nsl)], d_sl)
        dh.wait()
        magic = jnp.full((_L,), 0x5F3759DF, jnp.int32)

        @plsc.parallel_loop(0, nslv, unroll=4)
        def _(i):
            sl = pl.ds(i * _L, _L)
            deg = d_sl[sl] + 1.0
            y = plsc.bitcast(
                magic - lax.shift_right_logical(plsc.bitcast(deg, jnp.int32), 1),
                jnp.float32,
            )
            hd = -0.5 * deg
            y = y * (1.5 + hd * y * y)
            y = y * (1.5 + hd * y * y)
            y = y * (1.5 + hd * y * y)
            d_sl[sl] = y
            g_sl[sl] = y * h_sl[sl]

        pltpu.sync_copy(g_sl, g_sh.at[pl.ds(nbase, nsl)])

        @pl.when(c == 0)
        def _():
            pltpu.sync_copy(d_sl, dis_h.at[pl.ds(nbase, nsl)])
            pltpu.sync_copy(g_sl, g_h.at[pl.ds(nbase, nsl)])

        plsc.subcore_barrier()
        pltpu.sync_copy(g_sh, g_v)

        @plsc.parallel_loop(0, vecs, unroll=4)
        def _(i):
            sl = pl.ds(i * _L, _L)
            val_v[sl] = plsc.load_gather(g_v, [row1_v[sl]])

        dscs = pltpu.async_copy(val_v, acc_s.at[col_v], sem_sc, add=True)
        dscs.wait()
        dm.wait()
        ds2.wait()
        plsc.subcore_barrier()

        @pl.when(s == 0)
        def _():
            pltpu.sync_copy(acc_s, sp_h.at[c])

    return mk


def kernel(x, edge_index, split_n, We, be, Wg, bg):
    n, d = x.shape
    e = edge_index.shape[1]
    npad = _round_up(n, _NS * 128)
    ew = _round_up(e, _NW * 128) // _NW
    ep = ew * _NW

    p3 = pl.pallas_call(
        _make_proj_body(npad),
        out_shape=jax.ShapeDtypeStruct((3, npad), jnp.float32),
    )(x, We, Wg, be)

    eip = jnp.pad(edge_index, ((0, 0), (0, ep - e)), constant_values=n)

    eam, eas, sp, dis, g = _make_mega_kernel(ep, ew, npad)(
        eip, p3.reshape(3 * npad)
    )

    nm, ns, eamo, easo = pl.pallas_call(
        _make_soft_body(n, e),
        out_shape=[
            jax.ShapeDtypeStruct((n,), jnp.float32),
            jax.ShapeDtypeStruct((n,), jnp.float32),
            jax.ShapeDtypeStruct((e,), jnp.float32),
            jax.ShapeDtypeStruct((e,), jnp.float32),
        ],
    )(dis, g, sp, split_n, bg, eam, eas)

    return (
        eamo.reshape(e, 1),
        easo.reshape(e, 1),
        nm.reshape(n, 1),
        ns.reshape(n, 1),
    )
